# baseline (device time: 19746 ns/iter reference)
import jax
import jax.numpy as jnp
from jax import lax
from jax.experimental import pallas as pl
from jax.experimental.pallas import tpu as pltpu

B, H, D, BS = 8, 8, 64, 16
N_Y = 2
NEG = -1e30


def kernel(Q, K, V, bt, lens):
    p_loc = K.shape[0]
    keys_loc = p_loc * BS

    def body(q_ref, k_ref, v_ref, bt_ref, lens_ref, out_ref,
             u_scr, urem, send_sem, recv_sem):
        my_x = lax.axis_index("x")
        my_y = lax.axis_index("y")
        my_z = lax.axis_index("z")
        nbr = (my_x, 1 - my_y, my_z)

        barrier_sem = pltpu.get_barrier_semaphore()
        pl.semaphore_signal(
            barrier_sem, inc=1, device_id=nbr,
            device_id_type=pl.DeviceIdType.MESH,
        )

        btv = bt_ref[...]
        qsc = q_ref[...] * (D ** -0.5)

        page_of_key = (
            lax.broadcasted_iota(jnp.int32, (keys_loc, p_loc), 0) // BS
            + my_y * p_loc
        )
        jcol = lax.broadcasted_iota(jnp.int32, (keys_loc, p_loc), 1)
        cols = []
        for i in range(B):
            hit = (page_of_key == btv[i:i + 1, :]) & (jcol < lens_ref[i])
            cols.append(jnp.sum(jnp.where(hit, 1.0, 0.0), axis=1,
                                keepdims=True))
        ckey = jnp.concatenate(cols, axis=1)
        valid = ckey > 0.0

        for h in range(H):
            kh = k_ref[:, :, h, :].reshape(keys_loc, D)
            qh = qsc[:, 0, h, :]
            s = lax.dot_general(kh, qh, (((1,), (1,)), ((), ())),
                                preferred_element_type=jnp.float32)
            s = jnp.where(valid, s, NEG)
            m = jnp.max(s, axis=0, keepdims=True)
            e = ckey * jnp.exp(s - m)
            n = jnp.sum(e, axis=0, keepdims=True)
            vh = v_ref[:, :, h, :].reshape(keys_loc, D)
            u = lax.dot_general(e, vh, (((0,), (0,)), ((), ())),
                                preferred_element_type=jnp.float32)
            u_scr[h, :B, :] = u
            u_scr[h, B:B + 1, 0:B] = m
            u_scr[h, B:B + 1, B:2 * B] = n

        pl.semaphore_wait(barrier_sem, 1)
        rdma_u = pltpu.make_async_remote_copy(
            src_ref=u_scr, dst_ref=urem,
            send_sem=send_sem, recv_sem=recv_sem,
            device_id=nbr, device_id_type=pl.DeviceIdType.MESH,
        )
        rdma_u.start()
        rdma_u.wait()

        m0, n0 = u_scr[:, B, 0:B], u_scr[:, B, B:2 * B]
        m1, n1 = urem[:, B, 0:B], urem[:, B, B:2 * B]
        mx = jnp.maximum(m0, m1)
        w0 = jnp.exp(m0 - mx)
        w1 = jnp.exp(m1 - mx)
        num = (u_scr[:, :B, :] * w0[:, :, None]
               + urem[:, :B, :] * w1[:, :, None])
        den = n0 * w0 + n1 * w1
        out_ref[...] = num / den[:, :, None]

    out_t = pl.pallas_call(
        body,
        out_shape=jax.ShapeDtypeStruct((H, B, D), jnp.float32),
        in_specs=[
            pl.BlockSpec(memory_space=pltpu.VMEM),
            pl.BlockSpec(memory_space=pltpu.VMEM),
            pl.BlockSpec(memory_space=pltpu.VMEM),
            pl.BlockSpec(memory_space=pltpu.VMEM),
            pl.BlockSpec(memory_space=pltpu.SMEM),
        ],
        out_specs=pl.BlockSpec(memory_space=pltpu.VMEM),
        scratch_shapes=[
            pltpu.VMEM((H, B + 1, D), jnp.float32),
            pltpu.VMEM((H, B + 1, D), jnp.float32),
            pltpu.SemaphoreType.DMA(()),
            pltpu.SemaphoreType.DMA(()),
        ],
        compiler_params=pltpu.CompilerParams(collective_id=0),
    )(Q, K, V, bt, lens)

    return out_t.transpose(1, 0, 2).reshape(B, 1, H, D)
